# Initial kernel scaffold; baseline (speedup 1.0000x reference)
import functools

import jax
import jax.numpy as jnp
from jax import lax
from jax.experimental import pallas as pl
from jax.experimental.pallas import tpu as pltpu
from jax.experimental.pallas import tpu_sc as plsc

NC = 2
NS = 16
CH = 80


def _sc_partials(x, src, dst):
    N, D = x.shape
    E = src.shape[0]
    NW = NC * NS
    epw = E // NW
    assert epw * NW == E and epw % CH == 0 and epw % 8 == 0
    n_chunks = epw // CH
    rpt = N // NS
    assert rpt * NS == N

    mesh = plsc.VectorSubcoreMesh(core_axis_name="c", subcore_axis_name="s")

    @functools.partial(
        pl.kernel,
        out_type=jax.ShapeDtypeStruct((NC * N, D), jnp.float32),
        mesh=mesh,
        scratch_types=[
            pltpu.VMEM((CH,), jnp.int32),
            pltpu.VMEM((CH,), jnp.int32),
            pltpu.VMEM((CH, D), jnp.float32),
            pltpu.VMEM_SHARED((N, D), jnp.float32),
            pltpu.SemaphoreType.DMA,
        ],
    )
    def sc_kernel(x_hbm, src_hbm, dst_hbm, out_hbm, src_v, dst_v, rows_v,
                  acc, sem):
        cid = lax.axis_index("c")
        sid = lax.axis_index("s")
        wid = sid * NC + cid
        ebase = wid * epw
        rbase = sid * rpt

        pltpu.sync_copy(x_hbm.at[pl.ds(rbase, rpt)], acc.at[pl.ds(rbase, rpt)])
        plsc.subcore_barrier()

        def chunk(i, carry):
            cb = ebase + i * CH
            pltpu.sync_copy(src_hbm.at[pl.ds(cb, CH)], src_v)
            pltpu.sync_copy(dst_hbm.at[pl.ds(cb, CH)], dst_v)
            pltpu.async_copy(x_hbm.at[src_v], rows_v, sem).wait()
            pltpu.sync_copy(rows_v, acc.at[dst_v], add=True)
            return carry

        lax.fori_loop(0, n_chunks, chunk, 0)
        plsc.subcore_barrier()

        pltpu.sync_copy(acc.at[pl.ds(rbase, rpt)],
                        out_hbm.at[pl.ds(cid * N + rbase, rpt)])

    return sc_kernel(x, src, dst)


def _combine(partials, x):
    N, D = x.shape
    br = 2000
    grid = N // br
    assert br * grid == N

    def body(p0_ref, p1_ref, x_ref, o_ref):
        o_ref[...] = p0_ref[...] + p1_ref[...] - x_ref[...]

    return pl.pallas_call(
        body,
        grid=(grid,),
        in_specs=[
            pl.BlockSpec((br, D), lambda i: (i, 0)),
            pl.BlockSpec((br, D), lambda i: (i + grid, 0)),
            pl.BlockSpec((br, D), lambda i: (i, 0)),
        ],
        out_specs=pl.BlockSpec((br, D), lambda i: (i, 0)),
        out_shape=jax.ShapeDtypeStruct((N, D), jnp.float32),
    )(partials, partials, x)


def kernel(x, edge_index):
    src = edge_index[0]
    dst = edge_index[1]
    partials = _sc_partials(x, src, dst)
    return _combine(partials, x)


# speedup vs baseline: 5.5227x; 5.5227x over previous
import functools

import jax
import jax.numpy as jnp
from jax import lax
from jax.experimental import pallas as pl
from jax.experimental.pallas import tpu as pltpu
from jax.experimental.pallas import tpu_sc as plsc

NC = 2
NS = 16
CH = 80


def _sc_partials(x, src, dst):
    N, D = x.shape
    E = src.shape[0]
    NW = NC * NS
    epw = E // NW
    assert epw * NW == E and epw % CH == 0 and epw % 8 == 0
    n_chunks = epw // CH
    rpt = (N // NS) // 8 * 8
    rem = N - NS * rpt
    assert rem % 8 == 0 and rem >= 0

    mesh = plsc.VectorSubcoreMesh(core_axis_name="c", subcore_axis_name="s")

    @functools.partial(
        pl.kernel,
        out_type=jax.ShapeDtypeStruct((NC * N, D), jnp.float32),
        mesh=mesh,
        scratch_types=[
            pltpu.VMEM((CH,), jnp.int32),
            pltpu.VMEM((CH,), jnp.int32),
            pltpu.VMEM((CH, D), jnp.float32),
            pltpu.VMEM_SHARED((N, D), jnp.float32),
            pltpu.SemaphoreType.DMA,
        ],
    )
    def sc_kernel(x_hbm, src_hbm, dst_hbm, out_hbm, src_v, dst_v, rows_v,
                  acc, sem):
        cid = lax.axis_index("c")
        sid = lax.axis_index("s")
        wid = sid * NC + cid
        ebase = wid * epw
        rbase = sid * rpt

        pltpu.sync_copy(x_hbm.at[pl.ds(rbase, rpt)], acc.at[pl.ds(rbase, rpt)])
        if rem:
            @pl.when(sid == NS - 1)
            def _():
                pltpu.sync_copy(x_hbm.at[pl.ds(NS * rpt, rem)],
                                acc.at[pl.ds(NS * rpt, rem)])
        plsc.subcore_barrier()

        def chunk(i, carry):
            cb = ebase + i * CH
            pltpu.sync_copy(src_hbm.at[pl.ds(cb, CH)], src_v)
            pltpu.sync_copy(dst_hbm.at[pl.ds(cb, CH)], dst_v)
            pltpu.async_copy(x_hbm.at[src_v], rows_v, sem).wait()
            pltpu.sync_copy(rows_v, acc.at[dst_v], add=True)
            return carry

        lax.fori_loop(0, n_chunks, chunk, 0)
        plsc.subcore_barrier()

        pltpu.sync_copy(acc.at[pl.ds(rbase, rpt)],
                        out_hbm.at[pl.ds(cid * N + rbase, rpt)])
        if rem:
            @pl.when(sid == NS - 1)
            def _():
                pltpu.sync_copy(acc.at[pl.ds(NS * rpt, rem)],
                                out_hbm.at[pl.ds(cid * N + NS * rpt, rem)])

    return sc_kernel(x, src, dst)


def _combine(partials, x):
    N, D = x.shape
    br = 2000
    grid = N // br
    assert br * grid == N

    def body(p0_ref, p1_ref, x_ref, o_ref):
        o_ref[...] = p0_ref[...] + p1_ref[...] - x_ref[...]

    return pl.pallas_call(
        body,
        grid=(grid,),
        in_specs=[
            pl.BlockSpec((br, D), lambda i: (i, 0)),
            pl.BlockSpec((br, D), lambda i: (i + grid, 0)),
            pl.BlockSpec((br, D), lambda i: (i, 0)),
        ],
        out_specs=pl.BlockSpec((br, D), lambda i: (i, 0)),
        out_shape=jax.ShapeDtypeStruct((N, D), jnp.float32),
    )(partials, partials, x)


def kernel(x, edge_index):
    src = edge_index[0]
    dst = edge_index[1]
    partials = _sc_partials(x, src, dst)
    return _combine(partials, x)


# speedup vs baseline: 11.9038x; 2.1555x over previous
import functools

import jax
import jax.numpy as jnp
from jax import lax
from jax.experimental import pallas as pl
from jax.experimental.pallas import tpu as pltpu
from jax.experimental.pallas import tpu_sc as plsc

NC = 2
NS = 16
CH = 80
NB = 2


def _sc_partials(x, src, dst):
    N, D = x.shape
    E = src.shape[0]
    NW = NC * NS
    epw = E // NW
    assert epw * NW == E and epw % CH == 0 and epw % 8 == 0
    n_chunks = epw // CH
    assert n_chunks >= NB
    rpt = (N // NS) // 8 * 8
    rem = N - NS * rpt
    assert rem % 8 == 0 and rem >= 0

    mesh = plsc.VectorSubcoreMesh(core_axis_name="c", subcore_axis_name="s")

    @functools.partial(
        pl.kernel,
        out_type=jax.ShapeDtypeStruct((NC * N, D), jnp.float32),
        mesh=mesh,
        scratch_types=[
            pltpu.VMEM((epw,), jnp.int32),
            pltpu.VMEM((n_chunks, CH), jnp.int32),
            [pltpu.VMEM((CH, D), jnp.float32) for _ in range(NB)],
            pltpu.VMEM_SHARED((N, D), jnp.float32),
            [pltpu.SemaphoreType.DMA for _ in range(NB)],
        ],
    )
    def sc_kernel(x_hbm, src_hbm, dst_hbm, out_hbm, src_all, dst2d, rows,
                  acc, sems):
        cid = lax.axis_index("c")
        sid = lax.axis_index("s")
        wid = sid * NC + cid
        ebase = wid * epw
        rbase = sid * rpt

        pltpu.sync_copy(src_hbm.at[pl.ds(ebase, epw)], src_all)
        pltpu.sync_copy(dst_hbm.at[wid], dst2d)

        pltpu.sync_copy(x_hbm.at[pl.ds(rbase, rpt)], acc.at[pl.ds(rbase, rpt)])
        if rem:
            @pl.when(sid == NS - 1)
            def _():
                pltpu.sync_copy(x_hbm.at[pl.ds(NS * rpt, rem)],
                                acc.at[pl.ds(NS * rpt, rem)])
        plsc.subcore_barrier()

        for b in range(NB):
            pltpu.async_copy(x_hbm.at[src_all.at[pl.ds(b * CH, CH)]],
                             rows[b], sems[b])

        def outer(g, carry):
            for b in range(NB):
                i = g * NB + b
                pltpu.make_async_copy(x_hbm.at[pl.ds(0, CH)], rows[b],
                                      sems[b]).wait()
                pltpu.sync_copy(rows[b], acc.at[dst2d.at[i]], add=True)

                @pl.when(i + NB < n_chunks)
                def _():
                    pltpu.async_copy(
                        x_hbm.at[src_all.at[pl.ds((i + NB) * CH, CH)]],
                        rows[b], sems[b])
            return carry

        lax.fori_loop(0, n_chunks // NB, outer, 0)

        for r in range((n_chunks // NB) * NB, n_chunks):
            b = r % NB
            pltpu.make_async_copy(x_hbm.at[pl.ds(0, CH)], rows[b],
                                  sems[b]).wait()
            pltpu.sync_copy(rows[b], acc.at[dst2d.at[r]], add=True)

        plsc.subcore_barrier()

        pltpu.sync_copy(acc.at[pl.ds(rbase, rpt)],
                        out_hbm.at[pl.ds(cid * N + rbase, rpt)])
        if rem:
            @pl.when(sid == NS - 1)
            def _():
                pltpu.sync_copy(acc.at[pl.ds(NS * rpt, rem)],
                                out_hbm.at[pl.ds(cid * N + NS * rpt, rem)])

    return sc_kernel(x, src, dst.reshape(NW, n_chunks, CH))


def _combine(partials, x):
    N, D = x.shape
    br = 2000
    grid = N // br
    assert br * grid == N

    def body(p0_ref, p1_ref, x_ref, o_ref):
        o_ref[...] = p0_ref[...] + p1_ref[...] - x_ref[...]

    return pl.pallas_call(
        body,
        grid=(grid,),
        in_specs=[
            pl.BlockSpec((br, D), lambda i: (i, 0)),
            pl.BlockSpec((br, D), lambda i: (i + grid, 0)),
            pl.BlockSpec((br, D), lambda i: (i, 0)),
        ],
        out_specs=pl.BlockSpec((br, D), lambda i: (i, 0)),
        out_shape=jax.ShapeDtypeStruct((N, D), jnp.float32),
    )(partials, partials, x)


def kernel(x, edge_index):
    src = edge_index[0]
    dst = edge_index[1]
    partials = _sc_partials(x, src, dst)
    return _combine(partials, x)
